# baseline (speedup 1.0000x reference)
import functools

import numpy as np
import jax
import jax.numpy as jnp
from jax import lax
from jax.experimental import pallas as pl
from jax.experimental.pallas import tpu as pltpu
from jax.experimental.pallas import tpu_sc as plsc

VOCAB = 100000
D = 128
BATCH = 1024
SEQ = 200

NC = 2
NS = 16
NW = NC * NS
TOTAL = BATCH * SEQ
PER_W = TOTAL // NW
CHUNK = SEQ
NCHUNK = PER_W // CHUNK
G0 = 128
G1 = CHUNK - G0
GROUPS = D // 16
NBUF = 3


def _pos_table() -> np.ndarray:
    half = D // 2
    log_inc = np.log(10000.0) / (half - 1)
    inv = np.exp(-log_inc * np.arange(half, dtype=np.float32))
    t = np.arange(SEQ, dtype=np.float32)[:, None] * inv[None, :]
    return np.concatenate([np.sin(t), np.cos(t)], axis=1).astype(np.float32)


_POS = _pos_table()


def _sc_body(table_hbm, idx_hbm, pos_hbm, out_hbm,
             idx_v, pos_v, buf0, buf1, buf2,
             g0, g1, g2, s0, s1, s2, psem):
    bufs = (buf0, buf1, buf2)
    gsems = (g0, g1, g2)
    ssems = (s0, s1, s2)
    wid = lax.axis_index("s") * NC + lax.axis_index("c")
    base = wid * PER_W

    pltpu.sync_copy(idx_hbm.at[pl.ds(base, PER_W)], idx_v)
    pos_copy = pltpu.async_copy(pos_hbm, pos_v, psem)

    def start_gather(j, b):
        pltpu.async_copy(table_hbm.at[idx_v.at[pl.ds(j * CHUNK, CHUNK)]],
                         bufs[b], gsems[b])

    def wait_gather(j, b):
        pltpu.make_async_copy(table_hbm.at[idx_v.at[pl.ds(j * CHUNK, CHUNK)]],
                              bufs[b], gsems[b]).wait()

    def start_scatter(j, b):
        pltpu.async_copy(bufs[b], out_hbm.at[pl.ds(base + j * CHUNK, CHUNK)],
                         ssems[b])

    def wait_scatter(j, b):
        pltpu.make_async_copy(bufs[b],
                              out_hbm.at[pl.ds(base + j * CHUNK, CHUNK)],
                              ssems[b]).wait()

    def add_pos(b):
        buf = bufs[b]

        @plsc.parallel_loop(0, CHUNK, unroll=4)
        def row_body(r):
            for c in range(GROUPS):
                sl = pl.ds(c * 16, 16)
                plsc.addupdate(buf.at[r, sl], pos_v[r, sl])

    start_gather(0, 0)
    start_gather(1, 1)
    pos_copy.wait()

    wait_gather(0, 0)
    add_pos(0)
    start_scatter(0, 0)
    start_gather(2, 2)

    wait_gather(1, 1)
    add_pos(1)
    start_scatter(1, 1)
    wait_scatter(0, 0)
    start_gather(3, 0)

    def mid_body(t, carry):
        for bi in range(3):
            j = 2 + t * 3 + bi
            b = (2 + bi) % 3
            wait_gather(j, b)
            add_pos(b)
            start_scatter(j, b)
            wait_scatter(j - 1, (b + 2) % 3)
            start_gather(j + 2, (b + 2) % 3)
        return carry

    lax.fori_loop(0, 9, mid_body, 0)

    wait_gather(29, 2)
    add_pos(2)
    start_scatter(29, 2)
    wait_scatter(28, 1)
    start_gather(31, 1)

    wait_gather(30, 0)
    add_pos(0)
    start_scatter(30, 0)
    wait_scatter(29, 2)

    wait_gather(31, 1)
    add_pos(1)
    start_scatter(31, 1)
    wait_scatter(30, 0)
    wait_scatter(31, 1)


@functools.partial(jax.jit, static_argnames=())
def _sc_call(table, idx_flat, pos):
    mesh = plsc.VectorSubcoreMesh(
        core_axis_name="c", subcore_axis_name="s", num_cores=NC, num_subcores=NS
    )
    run = pl.kernel(
        _sc_body,
        out_type=jax.ShapeDtypeStruct((TOTAL, D), jnp.float32),
        mesh=mesh,
        scratch_types=[
            pltpu.VMEM((PER_W,), jnp.int32),
            pltpu.VMEM((SEQ, D), jnp.float32),
            pltpu.VMEM((CHUNK, D), jnp.float32),
            pltpu.VMEM((CHUNK, D), jnp.float32),
            pltpu.VMEM((CHUNK, D), jnp.float32),
            pltpu.SemaphoreType.DMA,
            pltpu.SemaphoreType.DMA,
            pltpu.SemaphoreType.DMA,
            pltpu.SemaphoreType.DMA,
            pltpu.SemaphoreType.DMA,
            pltpu.SemaphoreType.DMA,
            pltpu.SemaphoreType.DMA,
        ],
    )
    return run(table, idx_flat, pos)


def kernel(x, table):
    idx_flat = x.astype(jnp.int32).reshape(TOTAL)
    pos = jnp.asarray(_POS)
    out = _sc_call(table, idx_flat, pos)
    return out.reshape(BATCH, SEQ, D)


# speedup vs baseline: 1.0094x; 1.0094x over previous
import functools

import numpy as np
import jax
import jax.numpy as jnp
from jax import lax
from jax.experimental import pallas as pl
from jax.experimental.pallas import tpu as pltpu
from jax.experimental.pallas import tpu_sc as plsc

VOCAB = 100000
D = 128
BATCH = 1024
SEQ = 200

NC = 2
NS = 16
NW = NC * NS
TOTAL = BATCH * SEQ
PER_W = TOTAL // NW
CHUNK = SEQ
NCHUNK = PER_W // CHUNK
G0 = 128
G1 = CHUNK - G0
GROUPS = D // 16
NBUF = 3


def _pos_table() -> np.ndarray:
    half = D // 2
    log_inc = np.log(10000.0) / (half - 1)
    inv = np.exp(-log_inc * np.arange(half, dtype=np.float32))
    t = np.arange(SEQ, dtype=np.float32)[:, None] * inv[None, :]
    return np.concatenate([np.sin(t), np.cos(t)], axis=1).astype(np.float32)


_POS = _pos_table()


def _sc_body(table_hbm, idx_hbm, pos_hbm, out_hbm,
             idx_v, pos_v, buf0, buf1, buf2,
             g0, g1, g2, s0, s1, s2):
    bufs = (buf0, buf1, buf2)
    gsems = (g0, g1, g2)
    ssems = (s0, s1, s2)
    wid = lax.axis_index("s") * NC + lax.axis_index("c")
    base = wid * PER_W

    pltpu.sync_copy(idx_hbm.at[pl.ds(base, PER_W)], idx_v)
    pltpu.sync_copy(pos_hbm, pos_v)

    def start_gather(j, b):
        buf = bufs[b]
        pltpu.async_copy(table_hbm.at[idx_v.at[pl.ds(j * CHUNK, G0)]],
                         buf.at[pl.ds(0, G0)], gsems[b])
        pltpu.async_copy(table_hbm.at[idx_v.at[pl.ds(j * CHUNK + G0, G1)]],
                         buf.at[pl.ds(G0, G1)], gsems[b])

    def wait_gather(j, b):
        buf = bufs[b]
        pltpu.make_async_copy(table_hbm.at[idx_v.at[pl.ds(j * CHUNK, G0)]],
                              buf.at[pl.ds(0, G0)], gsems[b]).wait()
        pltpu.make_async_copy(table_hbm.at[idx_v.at[pl.ds(j * CHUNK + G0, G1)]],
                              buf.at[pl.ds(G0, G1)], gsems[b]).wait()

    def start_scatter(j, b):
        pltpu.async_copy(bufs[b], out_hbm.at[pl.ds(base + j * CHUNK, CHUNK)],
                         ssems[b])

    def wait_scatter(j, b):
        pltpu.make_async_copy(bufs[b],
                              out_hbm.at[pl.ds(base + j * CHUNK, CHUNK)],
                              ssems[b]).wait()

    def add_pos(b):
        buf = bufs[b]

        def row_body(r, carry):
            for c in range(GROUPS):
                sl = pl.ds(c * 16, 16)
                plsc.addupdate(buf.at[r, sl], pos_v[r, sl])
            return carry

        lax.fori_loop(0, CHUNK, row_body, 0)

    start_gather(0, 0)
    start_gather(1, 1)

    wait_gather(0, 0)
    add_pos(0)
    start_scatter(0, 0)
    start_gather(2, 2)

    wait_gather(1, 1)
    add_pos(1)
    start_scatter(1, 1)
    wait_scatter(0, 0)
    start_gather(3, 0)

    def mid_body(t, carry):
        for bi in range(3):
            j = 2 + t * 3 + bi
            b = (2 + bi) % 3
            wait_gather(j, b)
            add_pos(b)
            start_scatter(j, b)
            wait_scatter(j - 1, (b + 2) % 3)
            start_gather(j + 2, (b + 2) % 3)
        return carry

    lax.fori_loop(0, 9, mid_body, 0)

    wait_gather(29, 2)
    add_pos(2)
    start_scatter(29, 2)
    wait_scatter(28, 1)
    start_gather(31, 1)

    wait_gather(30, 0)
    add_pos(0)
    start_scatter(30, 0)
    wait_scatter(29, 2)

    wait_gather(31, 1)
    add_pos(1)
    start_scatter(31, 1)
    wait_scatter(30, 0)
    wait_scatter(31, 1)


@functools.partial(jax.jit, static_argnames=())
def _sc_call(table, idx_flat, pos):
    mesh = plsc.VectorSubcoreMesh(
        core_axis_name="c", subcore_axis_name="s", num_cores=NC, num_subcores=NS
    )
    run = pl.kernel(
        _sc_body,
        out_type=jax.ShapeDtypeStruct((TOTAL, D), jnp.float32),
        mesh=mesh,
        scratch_types=[
            pltpu.VMEM((PER_W,), jnp.int32),
            pltpu.VMEM((SEQ, D), jnp.float32),
            pltpu.VMEM((CHUNK, D), jnp.float32),
            pltpu.VMEM((CHUNK, D), jnp.float32),
            pltpu.VMEM((CHUNK, D), jnp.float32),
            pltpu.SemaphoreType.DMA,
            pltpu.SemaphoreType.DMA,
            pltpu.SemaphoreType.DMA,
            pltpu.SemaphoreType.DMA,
            pltpu.SemaphoreType.DMA,
            pltpu.SemaphoreType.DMA,
        ],
    )
    return run(table, idx_flat, pos)


def kernel(x, table):
    idx_flat = x.astype(jnp.int32).reshape(TOTAL)
    pos = jnp.asarray(_POS)
    out = _sc_call(table, idx_flat, pos)
    return out.reshape(BATCH, SEQ, D)
